# Initial kernel scaffold; baseline (speedup 1.0000x reference)
import jax
import jax.numpy as jnp
from jax.experimental import pallas as pl


def kernel(seg_maps, pixel_to_vertex_map, bary_coord_map):
    raise NotImplementedError("write your pallas kernel here")


# speedup vs baseline: 3.2231x; 3.2231x over previous
import functools

import jax
import jax.numpy as jnp
from jax import lax
from jax.experimental import pallas as pl
from jax.experimental.pallas import tpu as pltpu
from jax.experimental.pallas import tpu_sc as plsc

NV = 6890
NVP = 6912
THR = 0.3
B = 32
NC = 2
NS = 16
LANES = 16
NPIX = 4 * 256 * 256
PIX_PER_TILE = NPIX // NS
C_PX = 512
C_PC = 3 * C_PX
N_CHUNKS = PIX_PER_TILE // C_PX


def _sc_body(seg_hbm, vtx_hbm, bary_hbm, part_hbm, acc, idx_b, bc_b, seg_b):
    c = lax.axis_index("c")
    s = lax.axis_index("s")
    iota = lax.iota(jnp.int32, LANES)
    iota_seg = iota * C_PX
    iota_acc = iota * NVP
    zeros = jnp.zeros((LANES,), jnp.float32)

    def zero_cols(j, carry):
        for r in range(8):
            acc[pl.ds((j * 8 + r) * LANES, LANES)] = zeros
        return carry

    lax.fori_loop(0, NVP * LANES // (8 * LANES), zero_cols, 0)

    pc0 = s * (3 * PIX_PER_TILE)
    px0 = s * PIX_PER_TILE

    def chunk(k, carry):
        pltpu.sync_copy(vtx_hbm.at[pl.ds(pc0 + k * C_PC, C_PC)], idx_b)
        pltpu.sync_copy(bary_hbm.at[pl.ds(pc0 + k * C_PC, C_PC)], bc_b)
        for r in range(LANES):
            pltpu.sync_copy(
                seg_hbm.at[c * LANES + r, pl.ds(px0 + k * C_PX, C_PX)],
                seg_b.at[pl.ds(r * C_PX, C_PX)],
            )

        def pix(p, inner):
            segcol = plsc.load_gather(seg_b, [iota_seg + p])
            msk = segcol > THR
            for j in range(3):
                pcv = jnp.full((LANES,), p * 3 + j, jnp.int32)
                v = plsc.load_gather(idx_b, [pcv])
                bc = plsc.load_gather(bc_b, [pcv])
                plsc.addupdate_scatter(acc, [iota_acc + v], bc - THR, mask=msk)
            return inner

        lax.fori_loop(0, C_PX, pix, 0)
        return carry

    lax.fori_loop(0, N_CHUNKS, chunk, 0)

    pltpu.sync_copy(acc, part_hbm.at[s, c])


_sc_scatter = functools.partial(
    pl.kernel,
    out_type=jax.ShapeDtypeStruct((NS, NC, LANES * NVP), jnp.float32),
    mesh=plsc.VectorSubcoreMesh(
        core_axis_name="c", subcore_axis_name="s", num_cores=NC, num_subcores=NS
    ),
    scratch_types=[
        pltpu.VMEM((LANES * NVP,), jnp.float32),
        pltpu.VMEM((C_PC,), jnp.int32),
        pltpu.VMEM((C_PC,), jnp.float32),
        pltpu.VMEM((LANES * C_PX,), jnp.float32),
    ],
    compiler_params=pltpu.CompilerParams(needs_layout_passes=False),
)(_sc_body)


def _tc_body(part_ref, out_ref):
    ssum = jnp.sum(part_ref[...], axis=0)
    out_ref[...] = (ssum > 0.0).astype(jnp.float32)


_TC_BLK = 1152


def _tc_reduce(part):
    return pl.pallas_call(
        _tc_body,
        grid=(NVP // _TC_BLK,),
        in_specs=[pl.BlockSpec((NS, B, _TC_BLK), lambda i: (0, 0, i))],
        out_specs=pl.BlockSpec((B, _TC_BLK), lambda i: (0, i)),
        out_shape=jax.ShapeDtypeStruct((B, NVP), jnp.float32),
    )(part)


def kernel(seg_maps, pixel_to_vertex_map, bary_coord_map):
    seg2d = seg_maps.reshape(B, NPIX)
    vtx = pixel_to_vertex_map.reshape(-1)
    bc = bary_coord_map.reshape(-1)
    part = _sc_scatter(seg2d, vtx, bc)
    part = part.reshape(NS, B, NVP)
    out = _tc_reduce(part)
    return out[:, :NV]
